# baseline (speedup 1.0000x reference)
import functools

import jax
import jax.numpy as jnp
from jax import lax
from jax.experimental import pallas as pl
from jax.experimental.pallas import tpu as pltpu
from jax.experimental.pallas import tpu_sc as plsc

C = 128
K = 27
N_PAD = 100352
TILES = N_PAD // C
ZSPREAD = 16384
T_ROWS = N_PAD + ZSPREAD
T_TILES = T_ROWS // C
R = 2048


def _sc_gather(table, idx):
    info = plsc.get_sparse_core_info()
    nw = info.num_cores * info.num_subcores
    s_total = idx.shape[0]
    per_w = s_total // nw
    cols = table.shape[1]
    ch = 336
    bodies = per_w // (4 * ch)
    assert per_w * nw == s_total and 4 * ch * bodies == per_w

    mesh = plsc.VectorSubcoreMesh(core_axis_name="c", subcore_axis_name="s")

    @functools.partial(
        pl.kernel,
        mesh=mesh,
        out_type=jax.ShapeDtypeStruct((s_total, cols), table.dtype),
        scratch_types=[
            pltpu.VMEM((4 * ch,), jnp.int32),
            pltpu.VMEM((2, ch, cols), table.dtype),
            pltpu.SemaphoreType.DMA,
            pltpu.SemaphoreType.DMA,
            pltpu.SemaphoreType.DMA,
            pltpu.SemaphoreType.DMA,
        ],
    )
    def gather_kernel(table_hbm, idx_hbm, out_hbm, idx_v, rows_v,
                      g0, g1, w0, w1):
        cid = lax.axis_index("c")
        sid = lax.axis_index("s")
        wid = sid * info.num_cores + cid
        base = wid * per_w
        sem_g = (g0, g1)
        sem_w = (w0, w1)

        def chunk_off(body_ix, q):
            return base + body_ix * (4 * ch) + q * ch

        def body(body_ix, first):
            pltpu.sync_copy(idx_hbm.at[pl.ds(base + body_ix * 4 * ch, 4 * ch)],
                            idx_v)
            for q in range(4):
                s = q % 2
                off = chunk_off(body_ix, q)
                if not (first and q < 2):
                    pltpu.make_async_copy(rows_v.at[s],
                                          out_hbm.at[pl.ds(base, ch)],
                                          sem_w[s]).wait()
                pltpu.async_copy(table_hbm.at[idx_v.at[pl.ds(q * ch, ch)]],
                                 rows_v.at[s], sem_g[s])
                if q >= 1:
                    sp = (q - 1) % 2
                    poff = chunk_off(body_ix, q - 1)
                    pltpu.make_async_copy(table_hbm.at[idx_v.at[pl.ds(0, ch)]],
                                          rows_v.at[sp], sem_g[sp]).wait()
                    pltpu.async_copy(rows_v.at[sp],
                                     out_hbm.at[pl.ds(poff, ch)], sem_w[sp])
            pltpu.make_async_copy(table_hbm.at[idx_v.at[pl.ds(0, ch)]],
                                  rows_v.at[1], sem_g[1]).wait()
            pltpu.async_copy(rows_v.at[1],
                             out_hbm.at[pl.ds(chunk_off(body_ix, 3), ch)],
                             sem_w[1])

        body(0, True)
        lax.fori_loop(1, bodies, lambda j, c: (body(j, False), c)[1], 0)
        for s in range(2):
            pltpu.make_async_copy(rows_v.at[s], out_hbm.at[pl.ds(base, ch)],
                                  sem_w[s]).wait()

    return gather_kernel(table, idx)


def _pre_stage(feats, gamma1, beta1, n):
    in_blocks = (n + R - 1) // R

    def body(f_ref, g_ref, b_ref, o_ref):
        t = pl.program_id(0)
        x = f_ref[...]
        mu = jnp.mean(x, axis=-1, keepdims=True)
        var = jnp.mean((x - mu) ** 2, axis=-1, keepdims=True)
        y = (x - mu) * lax.rsqrt(var + 1e-5)
        y = y * g_ref[0] + b_ref[0]
        y = y * jax.nn.sigmoid(y)
        rows = t * R + lax.broadcasted_iota(jnp.int32, (R, C), 0)
        o_ref[...] = jnp.where(rows < n, y, 0.0)

    g2 = jnp.zeros((8, C), jnp.float32).at[0].set(gamma1)
    b2 = jnp.zeros((8, C), jnp.float32).at[0].set(beta1)
    return pl.pallas_call(
        body,
        grid=(T_ROWS // R,),
        in_specs=[
            pl.BlockSpec((R, C), lambda t: (jnp.minimum(t, in_blocks - 1), 0)),
            pl.BlockSpec((8, C), lambda t: (0, 0)),
            pl.BlockSpec((8, C), lambda t: (0, 0)),
        ],
        out_specs=pl.BlockSpec((R, C), lambda t: (t, 0)),
        out_shape=jax.ShapeDtypeStruct((T_ROWS, C), jnp.float32),
    )(feats, g2, b2)


def _mm1_stage(G, W1, b1, onehotB, shp, scp, n):
    def body(g_ref, w_ref, b_ref, oh_ref, sh_ref, sc_ref, f2_ref, acc_ref):
        t = pl.program_id(0)
        k = pl.program_id(1)

        @pl.when(k == 0)
        def _():
            acc_ref[...] = jnp.zeros_like(acc_ref)

        acc_ref[...] += jnp.dot(g_ref[0], w_ref[0], preferred_element_type=jnp.float32)

        @pl.when(k == K - 1)
        def _():
            y = acc_ref[...] + b_ref[0]
            mu = jnp.mean(y, axis=-1, keepdims=True)
            var = jnp.mean((y - mu) ** 2, axis=-1, keepdims=True)
            y = (y - mu) * lax.rsqrt(var + 1e-5)
            onehot = oh_ref[...]
            sh = jnp.dot(onehot, sh_ref[...], preferred_element_type=jnp.float32)
            sc = jnp.dot(onehot, sc_ref[...], preferred_element_type=jnp.float32)
            y = y * (1.0 + sc) + sh
            y = y * jax.nn.sigmoid(y)
            rows = t * R + lax.broadcasted_iota(jnp.int32, (R, C), 0)
            f2_ref[...] = jnp.where(rows < n, y, 0.0)

    b2 = jnp.zeros((8, C), jnp.float32).at[0].set(b1)
    nblk = N_PAD // R
    return pl.pallas_call(
        body,
        grid=(T_ROWS // R, K),
        in_specs=[
            pl.BlockSpec((1, R, C), lambda t, k: (k, jnp.minimum(t, nblk - 1), 0)),
            pl.BlockSpec((1, C, C), lambda t, k: (k, 0, 0)),
            pl.BlockSpec((8, C), lambda t, k: (0, 0)),
            pl.BlockSpec((R, 8), lambda t, k: (t, 0)),
            pl.BlockSpec((8, C), lambda t, k: (0, 0)),
            pl.BlockSpec((8, C), lambda t, k: (0, 0)),
        ],
        out_specs=pl.BlockSpec((R, C), lambda t, k: (t, 0)),
        out_shape=jax.ShapeDtypeStruct((T_ROWS, C), jnp.float32),
        scratch_shapes=[pltpu.VMEM((R, C), jnp.float32)],
    )(G, W1, b2, onehotB, shp, scp)


def _mm2_stage(G, W2, b2):
    def body(g_ref, w_ref, b_ref, o_ref):
        k = pl.program_id(1)

        @pl.when(k == 0)
        def _():
            o_ref[...] = jnp.zeros_like(o_ref)

        o_ref[...] += jnp.dot(g_ref[0], w_ref[0], preferred_element_type=jnp.float32)

        @pl.when(k == K - 1)
        def _():
            o_ref[...] += b_ref[0]

    b2p = jnp.zeros((8, C), jnp.float32).at[0].set(b2)
    return pl.pallas_call(
        body,
        grid=(N_PAD // R, K),
        in_specs=[
            pl.BlockSpec((1, R, C), lambda t, k: (k, t, 0)),
            pl.BlockSpec((1, C, C), lambda t, k: (k, 0, 0)),
            pl.BlockSpec((8, C), lambda t, k: (0, 0)),
        ],
        out_specs=pl.BlockSpec((R, C), lambda t, k: (t, 0)),
        out_shape=jax.ShapeDtypeStruct((N_PAD, C), jnp.float32),
    )(G, W2, b2p)


TOT = K * N_PAD
HALF = TOT // 2
TRASHN = 65536
SCH = 2048
PCH = 8192
PSUB = HALF // 16
PTAIL = PSUB - 10 * PCH


def _sc_scatter(a0, keys2, vals, m8):
    nch = -(-m8 // SCH)
    nt = -(-nch // 16)
    mesh = plsc.VectorSubcoreMesh(core_axis_name="c", subcore_axis_name="s")

    @functools.partial(
        pl.kernel,
        mesh=mesh,
        out_type=jax.ShapeDtypeStruct((TOT + TRASHN,), jnp.int32),
        scratch_types=[
            pltpu.VMEM((PCH,), jnp.int32),
            pltpu.VMEM((SCH,), jnp.int32),
            pltpu.VMEM((SCH,), jnp.int32),
        ],
    )
    def scat(a0_hbm, keys_hbm, vals_hbm, out_hbm, buf, key_v, val_v):
        cid = lax.axis_index("c")
        sid = lax.axis_index("s")
        base = cid * HALF + sid * PSUB
        for i in range(10):
            off = base + i * PCH
            pltpu.sync_copy(a0_hbm.at[pl.ds(off, PCH)], buf)
            pltpu.sync_copy(buf, out_hbm.at[pl.ds(off, PCH)])
        off = base + 10 * PCH
        pltpu.sync_copy(a0_hbm.at[pl.ds(off, PTAIL)], buf.at[pl.ds(0, PTAIL)])
        pltpu.sync_copy(buf.at[pl.ds(0, PTAIL)], out_hbm.at[pl.ds(off, PTAIL)])
        plsc.subcore_barrier()
        for t in range(nt):
            start = jnp.minimum((sid + t * 16) * SCH, m8 - SCH)
            pltpu.sync_copy(keys_hbm.at[pl.ds(cid * m8 + start, SCH)], key_v)
            pltpu.sync_copy(vals_hbm.at[pl.ds(start, SCH)], val_v)
            pltpu.sync_copy(val_v, out_hbm.at[key_v])

    return scat(a0, keys2, vals)


def _build_amap(in_idx, out_idx, counts):
    m = in_idx.shape[0]
    m8 = m + (-m) % 8
    ends = jnp.cumsum(jnp.asarray(counts).astype(jnp.int32))
    k_e = jnp.searchsorted(ends, jnp.arange(m, dtype=jnp.int32),
                           side="right").astype(jnp.int32)
    key = jnp.pad(k_e * N_PAD + out_idx.astype(jnp.int32), (0, m8 - m),
                  mode="edge")
    vals = jnp.pad(in_idx.astype(jnp.int32), (0, m8 - m), mode="edge")
    trash = TOT + jnp.arange(m8, dtype=jnp.int32) % TRASHN
    keys2 = jnp.concatenate([jnp.where(key < HALF, key, trash),
                             jnp.where(key >= HALF, key, trash)])
    a0 = N_PAD + jnp.arange(TOT, dtype=jnp.int32) % ZSPREAD
    return _sc_scatter(a0, keys2, vals, m8)[:TOT]


def kernel(feats, shift_3d, scale_3d, gamma1, beta1, W1, b1, W2, b2,
           batch_idx, in1, out1, counts1, in2, out2, counts2):
    n = feats.shape[0]
    a1 = _build_amap(in1, out1, counts1)
    a2 = _build_amap(in2, out2, counts2)

    bidx_pad = jnp.zeros((T_ROWS,), jnp.int32).at[:n].set(batch_idx.astype(jnp.int32))
    onehotB = (bidx_pad[:, None] == jnp.arange(8, dtype=jnp.int32)[None, :]
               ).astype(jnp.float32)
    shp = jnp.zeros((8, C), jnp.float32).at[:4].set(shift_3d)
    scp = jnp.zeros((8, C), jnp.float32).at[:4].set(scale_3d)

    f1 = _pre_stage(feats, gamma1, beta1, n)
    g1 = _sc_gather(f1, a1).reshape(K, N_PAD, C)
    f2 = _mm1_stage(g1, W1, b1, onehotB, shp, scp, n)
    g2 = _sc_gather(f2, a2).reshape(K, N_PAD, C)
    out = _mm2_stage(g2, W2, b2)
    return out[:n]


# speedup vs baseline: 1.1242x; 1.1242x over previous
import functools

import jax
import jax.numpy as jnp
from jax import lax
from jax.experimental import pallas as pl
from jax.experimental.pallas import tpu as pltpu
from jax.experimental.pallas import tpu_sc as plsc

C = 128
K = 27
N_PAD = 100352
TILES = N_PAD // C
ZSPREAD = 16384
T_ROWS = N_PAD + ZSPREAD
T_TILES = T_ROWS // C
R = 2048


def _sc_gather(table, idx):
    info = plsc.get_sparse_core_info()
    nw = info.num_cores * info.num_subcores
    s_total = idx.shape[0]
    per_w = s_total // nw
    cols = table.shape[1]
    ch = 336
    bodies = per_w // (4 * ch)
    assert per_w * nw == s_total and 4 * ch * bodies == per_w

    mesh = plsc.VectorSubcoreMesh(core_axis_name="c", subcore_axis_name="s")

    @functools.partial(
        pl.kernel,
        mesh=mesh,
        out_type=jax.ShapeDtypeStruct((s_total, cols), table.dtype),
        scratch_types=[
            pltpu.VMEM((4 * ch,), jnp.int32),
            pltpu.VMEM((2, ch, cols), table.dtype),
            pltpu.SemaphoreType.DMA,
            pltpu.SemaphoreType.DMA,
            pltpu.SemaphoreType.DMA,
            pltpu.SemaphoreType.DMA,
        ],
    )
    def gather_kernel(table_hbm, idx_hbm, out_hbm, idx_v, rows_v,
                      g0, g1, w0, w1):
        cid = lax.axis_index("c")
        sid = lax.axis_index("s")
        wid = sid * info.num_cores + cid
        base = wid * per_w
        sem_g = (g0, g1)
        sem_w = (w0, w1)

        def chunk_off(body_ix, q):
            return base + body_ix * (4 * ch) + q * ch

        def body(body_ix, first):
            pltpu.sync_copy(idx_hbm.at[pl.ds(base + body_ix * 4 * ch, 4 * ch)],
                            idx_v)
            for q in range(4):
                s = q % 2
                off = chunk_off(body_ix, q)
                if not (first and q < 2):
                    pltpu.make_async_copy(rows_v.at[s],
                                          out_hbm.at[pl.ds(base, ch)],
                                          sem_w[s]).wait()
                pltpu.async_copy(table_hbm.at[idx_v.at[pl.ds(q * ch, ch)]],
                                 rows_v.at[s], sem_g[s])
                if q >= 1:
                    sp = (q - 1) % 2
                    poff = chunk_off(body_ix, q - 1)
                    pltpu.make_async_copy(table_hbm.at[idx_v.at[pl.ds(0, ch)]],
                                          rows_v.at[sp], sem_g[sp]).wait()
                    pltpu.async_copy(rows_v.at[sp],
                                     out_hbm.at[pl.ds(poff, ch)], sem_w[sp])
            pltpu.make_async_copy(table_hbm.at[idx_v.at[pl.ds(0, ch)]],
                                  rows_v.at[1], sem_g[1]).wait()
            pltpu.async_copy(rows_v.at[1],
                             out_hbm.at[pl.ds(chunk_off(body_ix, 3), ch)],
                             sem_w[1])

        body(0, True)
        lax.fori_loop(1, bodies, lambda j, c: (body(j, False), c)[1], 0)
        for s in range(2):
            pltpu.make_async_copy(rows_v.at[s], out_hbm.at[pl.ds(base, ch)],
                                  sem_w[s]).wait()

    return gather_kernel(table, idx)


def _pre_stage(feats, gamma1, beta1, n):
    in_blocks = (n + R - 1) // R

    def body(f_ref, g_ref, b_ref, o_ref):
        t = pl.program_id(0)
        x = f_ref[...]
        mu = jnp.mean(x, axis=-1, keepdims=True)
        var = jnp.mean((x - mu) ** 2, axis=-1, keepdims=True)
        y = (x - mu) * lax.rsqrt(var + 1e-5)
        y = y * g_ref[0] + b_ref[0]
        y = y * jax.nn.sigmoid(y)
        rows = t * R + lax.broadcasted_iota(jnp.int32, (R, C), 0)
        o_ref[...] = jnp.where(rows < n, y, 0.0)

    g2 = jnp.zeros((8, C), jnp.float32).at[0].set(gamma1)
    b2 = jnp.zeros((8, C), jnp.float32).at[0].set(beta1)
    return pl.pallas_call(
        body,
        grid=(T_ROWS // R,),
        in_specs=[
            pl.BlockSpec((R, C), lambda t: (jnp.minimum(t, in_blocks - 1), 0)),
            pl.BlockSpec((8, C), lambda t: (0, 0)),
            pl.BlockSpec((8, C), lambda t: (0, 0)),
        ],
        out_specs=pl.BlockSpec((R, C), lambda t: (t, 0)),
        out_shape=jax.ShapeDtypeStruct((T_ROWS, C), jnp.float32),
    )(feats, g2, b2)


def _mm1_stage(G, W1, b1, onehotB, shp, scp, n):
    def body(g_ref, w_ref, b_ref, oh_ref, sh_ref, sc_ref, f2_ref, acc_ref):
        t = pl.program_id(0)
        k = pl.program_id(1)

        @pl.when(k == 0)
        def _():
            acc_ref[...] = jnp.zeros_like(acc_ref)

        acc_ref[...] += jnp.dot(g_ref[0], w_ref[0], preferred_element_type=jnp.float32)

        @pl.when(k == K - 1)
        def _():
            y = acc_ref[...] + b_ref[0]
            mu = jnp.mean(y, axis=-1, keepdims=True)
            var = jnp.mean((y - mu) ** 2, axis=-1, keepdims=True)
            y = (y - mu) * lax.rsqrt(var + 1e-5)
            onehot = oh_ref[...]
            sh = jnp.dot(onehot, sh_ref[...], preferred_element_type=jnp.float32)
            sc = jnp.dot(onehot, sc_ref[...], preferred_element_type=jnp.float32)
            y = y * (1.0 + sc) + sh
            y = y * jax.nn.sigmoid(y)
            rows = t * R + lax.broadcasted_iota(jnp.int32, (R, C), 0)
            f2_ref[...] = jnp.where(rows < n, y, 0.0)

    b2 = jnp.zeros((8, C), jnp.float32).at[0].set(b1)
    nblk = N_PAD // R
    return pl.pallas_call(
        body,
        grid=(T_ROWS // R, K),
        in_specs=[
            pl.BlockSpec((1, R, C), lambda t, k: (k, jnp.minimum(t, nblk - 1), 0)),
            pl.BlockSpec((1, C, C), lambda t, k: (k, 0, 0)),
            pl.BlockSpec((8, C), lambda t, k: (0, 0)),
            pl.BlockSpec((R, 8), lambda t, k: (t, 0)),
            pl.BlockSpec((8, C), lambda t, k: (0, 0)),
            pl.BlockSpec((8, C), lambda t, k: (0, 0)),
        ],
        out_specs=pl.BlockSpec((R, C), lambda t, k: (t, 0)),
        out_shape=jax.ShapeDtypeStruct((T_ROWS, C), jnp.float32),
        scratch_shapes=[pltpu.VMEM((R, C), jnp.float32)],
    )(G, W1, b2, onehotB, shp, scp)


def _mm2_stage(G, W2, b2):
    def body(g_ref, w_ref, b_ref, o_ref):
        k = pl.program_id(1)

        @pl.when(k == 0)
        def _():
            o_ref[...] = jnp.zeros_like(o_ref)

        o_ref[...] += jnp.dot(g_ref[0], w_ref[0], preferred_element_type=jnp.float32)

        @pl.when(k == K - 1)
        def _():
            o_ref[...] += b_ref[0]

    b2p = jnp.zeros((8, C), jnp.float32).at[0].set(b2)
    return pl.pallas_call(
        body,
        grid=(N_PAD // R, K),
        in_specs=[
            pl.BlockSpec((1, R, C), lambda t, k: (k, t, 0)),
            pl.BlockSpec((1, C, C), lambda t, k: (k, 0, 0)),
            pl.BlockSpec((8, C), lambda t, k: (0, 0)),
        ],
        out_specs=pl.BlockSpec((R, C), lambda t, k: (t, 0)),
        out_shape=jax.ShapeDtypeStruct((N_PAD, C), jnp.float32),
    )(G, W2, b2p)


TOT = K * N_PAD


def _build_amap(in_idx, out_idx, counts):
    m = in_idx.shape[0]
    ends = jnp.cumsum(jnp.asarray(counts).astype(jnp.int32))
    k_e = jnp.searchsorted(ends, jnp.arange(m, dtype=jnp.int32),
                           side="right").astype(jnp.int32)
    a = N_PAD + jnp.arange(TOT, dtype=jnp.int32) % ZSPREAD
    key = k_e * N_PAD + out_idx.astype(jnp.int32)
    return a.at[key].set(in_idx.astype(jnp.int32),
                         indices_are_sorted=True, unique_indices=True)


def kernel(feats, shift_3d, scale_3d, gamma1, beta1, W1, b1, W2, b2,
           batch_idx, in1, out1, counts1, in2, out2, counts2):
    n = feats.shape[0]
    a1 = _build_amap(in1, out1, counts1)
    a2 = _build_amap(in2, out2, counts2)

    bidx_pad = jnp.zeros((T_ROWS,), jnp.int32).at[:n].set(batch_idx.astype(jnp.int32))
    onehotB = (bidx_pad[:, None] == jnp.arange(8, dtype=jnp.int32)[None, :]
               ).astype(jnp.float32)
    shp = jnp.zeros((8, C), jnp.float32).at[:4].set(shift_3d)
    scp = jnp.zeros((8, C), jnp.float32).at[:4].set(scale_3d)

    f1 = _pre_stage(feats, gamma1, beta1, n)
    g1 = _sc_gather(f1, a1).reshape(K, N_PAD, C)
    f2 = _mm1_stage(g1, W1, b1, onehotB, shp, scp, n)
    g2 = _sc_gather(f2, a2).reshape(K, N_PAD, C)
    out = _mm2_stage(g2, W2, b2)
    return out[:n]
